# Initial kernel scaffold; baseline (speedup 1.0000x reference)
import jax
import jax.numpy as jnp
from jax.experimental import pallas as pl


def kernel(input1, input2, eps, iters):
    raise NotImplementedError("write your pallas kernel here")


# speedup vs baseline: 1.2769x; 1.2769x over previous
import functools

import jax
import jax.numpy as jnp
from jax import lax
from jax.experimental import pallas as pl
from jax.experimental.pallas import tpu as pltpu
from jax.experimental.pallas import tpu_sc as plsc

_L = 16
_NC = 2
_NS = 16
_NW = _NC * _NS


def _nn_body(n, qpw, x1_hbm, x2_hbm, minsq_hbm, arg_hbm,
             kx_v, ky_v, kz_v, qx_v, qy_v, qz_v, om_v, oa_v):
    tiles_per_b = n // qpw
    wid = lax.axis_index("s") * _NC + lax.axis_index("c")
    b = wid // tiles_per_b
    qbase = (wid % tiles_per_b) * qpw

    kofs = b * (3 * n)
    qofs = kofs + qbase
    pltpu.sync_copy(x2_hbm.at[pl.ds(kofs, n)], kx_v)
    pltpu.sync_copy(x2_hbm.at[pl.ds(kofs + n, n)], ky_v)
    pltpu.sync_copy(x2_hbm.at[pl.ds(kofs + 2 * n, n)], kz_v)
    pltpu.sync_copy(x1_hbm.at[pl.ds(qofs, qpw)], qx_v)
    pltpu.sync_copy(x1_hbm.at[pl.ds(qofs + n, qpw)], qy_v)
    pltpu.sync_copy(x1_hbm.at[pl.ds(qofs + 2 * n, qpw)], qz_v)

    def group(g, _):
        qx = qx_v[pl.ds(g * _L, _L)]
        qy = qy_v[pl.ds(g * _L, _L)]
        qz = qz_v[pl.ds(g * _L, _L)]

        def scan_chunk(c, carry):
            run_min, run_arg = carry
            base = c * _L
            kxc = kx_v[pl.ds(base, _L)]
            kyc = ky_v[pl.ds(base, _L)]
            kzc = kz_v[pl.ds(base, _L)]
            for u in range(_L):
                sel = jnp.full((_L,), u, jnp.int32)
                dx = qx - kxc.at[sel].get(mode="promise_in_bounds")
                dy = qy - kyc.at[sel].get(mode="promise_in_bounds")
                dz = qz - kzc.at[sel].get(mode="promise_in_bounds")
                d2 = dx * dx + dy * dy + dz * dz
                better = d2 < run_min
                run_min = jnp.where(better, d2, run_min)
                run_arg = jnp.where(better,
                                    jnp.full((_L,), base + u, jnp.int32),
                                    run_arg)
            return run_min, run_arg

        init = (jnp.full((_L,), jnp.inf, jnp.float32),
                jnp.zeros((_L,), jnp.int32))
        run_min, run_arg = lax.fori_loop(0, n // _L, scan_chunk, init)
        om_v[pl.ds(g * _L, _L)] = run_min
        oa_v[pl.ds(g * _L, _L)] = run_arg
        return _

    lax.fori_loop(0, qpw // _L, group, 0)

    obase = b * n + qbase
    pltpu.sync_copy(om_v, minsq_hbm.at[pl.ds(obase, qpw)])
    pltpu.sync_copy(oa_v, arg_hbm.at[pl.ds(obase, qpw)])


@functools.partial(jax.jit, static_argnums=(2, 3))
def _nn_sc(x1t, x2t, B, n):
    qpw = (B * n) // _NW
    mesh = plsc.VectorSubcoreMesh(core_axis_name="c", subcore_axis_name="s")
    body = functools.partial(_nn_body, n, qpw)
    ker = pl.kernel(
        body,
        out_type=[jax.ShapeDtypeStruct((B * n,), jnp.float32),
                  jax.ShapeDtypeStruct((B * n,), jnp.int32)],
        mesh=mesh,
        scratch_types=[
            pltpu.VMEM((n,), jnp.float32),
            pltpu.VMEM((n,), jnp.float32),
            pltpu.VMEM((n,), jnp.float32),
            pltpu.VMEM((qpw,), jnp.float32),
            pltpu.VMEM((qpw,), jnp.float32),
            pltpu.VMEM((qpw,), jnp.float32),
            pltpu.VMEM((qpw,), jnp.float32),
            pltpu.VMEM((qpw,), jnp.int32),
        ],
    )
    minsq, arg = ker(x1t, x2t)
    return minsq.reshape(B, n), arg.reshape(B, n)


def kernel(input1, input2, eps, iters):
    B, n, _ = input1.shape
    x1t = jnp.transpose(input1, (0, 2, 1)).reshape(-1)
    x2t = jnp.transpose(input2, (0, 2, 1)).reshape(-1)
    minsq, arg = _nn_sc(x1t, x2t, B, n)

    m = jnp.sqrt(minsq)

    def body(_, carry):
        price, _mc = carry
        mc = m - price
        return price + eps * mc, mc

    _price, mc = lax.fori_loop(
        0, iters, body, (jnp.zeros_like(m), jnp.zeros_like(m)))
    arg = jnp.where(iters >= 1, arg, jnp.zeros_like(arg))
    return jnp.sqrt(mc), arg
